# baseline (speedup 1.0000x reference)
import jax
import jax.numpy as jnp
from jax import lax
from jax.experimental import pallas as pl
from jax.experimental.pallas import tpu as pltpu
from jax.experimental.pallas import tpu_sc as plsc

N = 10000
E = 320000
D = 128
L = 3
B = 64
R = 16

NCORE = 2
NQ = 4
DQ = D // NQ
DH = D // NCORE
NT = 16
CH = 128
CPT = 158
EPT = CH * CPT
EPAD = NT * EPT
NP = 10240
STRIPE = NP // NT
ZCH = 128

_MESH = plsc.VectorSubcoreMesh(core_axis_name="c", subcore_axis_name="s")


def _make_edge_pass(with_deg):
  out_type = [jax.ShapeDtypeStruct((NQ, NP, DQ), jnp.float32)]
  scratch = [
      pltpu.VMEM((CPT, CH), jnp.int32),
      pltpu.VMEM((CPT, CH), jnp.int32),
      pltpu.VMEM((CH, DQ), jnp.float32),
      pltpu.VMEM((CH, DQ), jnp.float32),
      pltpu.VMEM((ZCH, DQ), jnp.float32),
      pltpu.VMEM_SHARED((NP, DQ), jnp.float32),
      pltpu.VMEM_SHARED((NP, DQ), jnp.float32),
      pltpu.SemaphoreType.DMA,
      pltpu.SemaphoreType.DMA,
      pltpu.SemaphoreType.DMA,
      pltpu.SemaphoreType.DMA,
  ]
  if with_deg:
    out_type.append(jax.ShapeDtypeStruct((NCORE, NP, 16), jnp.float32))
    scratch += [
        pltpu.VMEM((CH, 16), jnp.float32),
        pltpu.VMEM((ZCH, 16), jnp.float32),
        pltpu.VMEM_SHARED((NP, 16), jnp.float32),
    ]

  def body(*args):
    if with_deg:
      (hq, src_idx, dst_idx, zrows, ones_hbm, z16_hbm,
       agg_out, deg_out,
       src_v, dst_v, rows0, rows1, zero_v, hsp, acc, g0, g1, s0, s1,
       ones_v, z16_v, dacc) = args
    else:
      (hq, src_idx, dst_idx, zrows,
       agg_out,
       src_v, dst_v, rows0, rows1, zero_v, hsp, acc, g0, g1, s0, s1) = args
    c = lax.axis_index("c")
    s = lax.axis_index("s")
    pltpu.sync_copy(zrows, zero_v)
    pltpu.sync_copy(src_idx.at[s], src_v)
    pltpu.sync_copy(dst_idx.at[s], dst_v)
    if with_deg:
      pltpu.sync_copy(ones_hbm, ones_v)
      pltpu.sync_copy(z16_hbm, z16_v)
    base = s * STRIPE
    for p in range(2):
      q = 2 * c + p
      pltpu.sync_copy(hq.at[q, pl.ds(base, STRIPE)],
                      hsp.at[pl.ds(base, STRIPE)])
      for k in range(STRIPE // ZCH):
        pltpu.sync_copy(zero_v, acc.at[pl.ds(base + k * ZCH, ZCH)])
        if with_deg and p == 0:
          pltpu.sync_copy(z16_v, dacc.at[pl.ds(base + k * ZCH, ZCH)])
      plsc.subcore_barrier()

      pltpu.async_copy(hsp.at[src_v.at[0]], rows0, g0)

      def pair(jj, carry):
        a = 2 * jj
        b = a + 1
        nxt = jnp.where(b + 1 >= CPT, 0, b + 1)
        pltpu.async_copy(hsp.at[src_v.at[b]], rows1, g1)
        pltpu.make_async_copy(hsp.at[src_v.at[a]], rows0, g0).wait()
        pltpu.async_copy(rows0, acc.at[dst_v.at[a]], s0, add=True)
        if with_deg and p == 0:
          @pl.when(c == 0)
          def _():
            pltpu.sync_copy(ones_v, dacc.at[dst_v.at[a]], add=True)

          @pl.when(c == 1)
          def _():
            pltpu.sync_copy(ones_v, dacc.at[dst_v.at[b]], add=True)
        pltpu.make_async_copy(hsp.at[src_v.at[b]], rows1, g1).wait()
        pltpu.async_copy(rows1, acc.at[dst_v.at[b]], s1, add=True)
        pltpu.make_async_copy(rows0, acc.at[dst_v.at[a]], s0).wait()
        pltpu.async_copy(hsp.at[src_v.at[nxt]], rows0, g0)
        pltpu.make_async_copy(rows1, acc.at[dst_v.at[b]], s1).wait()
        return carry

      lax.fori_loop(0, CPT // 2, pair, 0)
      pltpu.make_async_copy(hsp.at[src_v.at[0]], rows0, g0).wait()
      plsc.subcore_barrier()
      pltpu.sync_copy(acc.at[pl.ds(base, STRIPE)],
                      agg_out.at[q, pl.ds(base, STRIPE)])
      if with_deg and p == 0:
        pltpu.sync_copy(dacc.at[pl.ds(base, STRIPE)],
                        deg_out.at[c, pl.ds(base, STRIPE)])

  return pl.kernel(body, out_type=tuple(out_type), mesh=_MESH,
                   scratch_types=tuple(scratch),
                   compiler_params=pltpu.CompilerParams(
                       use_tc_tiling_on_sc=False))


_edge_pass_deg = _make_edge_pass(True)
_edge_pass = _make_edge_pass(False)


BLK = 512
NB = NP // BLK


def _make_layer(with_cnt, with_max):
  out_shape = [jax.ShapeDtypeStruct((NQ, NP, DQ), jnp.float32),
               jax.ShapeDtypeStruct((B, D), jnp.float32)]
  out_specs = [pl.BlockSpec((NQ, BLK, DQ), lambda i: (0, i, 0)),
               pl.BlockSpec((B, D), lambda i: (0, 0))]
  if with_cnt:
    out_shape.append(jax.ShapeDtypeStruct((B, 128), jnp.float32))
    out_specs.append(pl.BlockSpec((B, 128), lambda i: (0, 0)))
  if with_max:
    out_shape.append(jax.ShapeDtypeStruct((B, D), jnp.float32))
    out_specs.append(pl.BlockSpec((B, D), lambda i: (0, 0)))

  def body(*refs):
    (a0_ref, a1_ref, a2_ref, a3_ref, deg_ref, w_ref, gidr_ref, gidc_ref,
     gids_ref, hh_ref, sum_ref, *rest) = refs
    cnt_ref = rest[0] if with_cnt else None
    mx_ref = rest[1 if with_cnt else 0] if with_max else None

    i = pl.program_id(0)
    agg = jnp.concatenate([a0_ref[0], a1_ref[0], a2_ref[0], a3_ref[0]],
                          axis=1)
    deg = deg_ref[0, :, 0:1] + deg_ref[1, :, 0:1]
    norm = 1.0 / jnp.maximum(deg, 1.0)
    h = jnp.dot(agg * norm, w_ref[...], preferred_element_type=jnp.float32)
    h = jnp.maximum(h, 0.0)
    for q in range(NQ):
      hh_ref[q] = h[:, q * DQ:(q + 1) * DQ]

    @pl.when(i == 0)
    def _():
      sum_ref[...] = jnp.zeros_like(sum_ref)
      if with_cnt:
        cnt_ref[...] = jnp.zeros_like(cnt_ref)
      if with_max:
        mx_ref[...] = jnp.zeros_like(mx_ref)

    gid_row = gidr_ref[0]
    M = (gid_row == lax.broadcasted_iota(jnp.int32, (B, BLK), 0)
         ).astype(jnp.float32)
    sum_ref[...] += jnp.dot(M, h, preferred_element_type=jnp.float32)
    if with_cnt:
      cnt_ref[...] += jnp.dot(M, jnp.ones((BLK, 128), jnp.float32),
                              preferred_element_type=jnp.float32)
    if with_max:
      gid_col = gidc_ref[0]

      def bstep(b, cur):
        mrow = jnp.max(jnp.where(gid_col == b, h, 0.0), axis=0,
                       keepdims=True)
        sel = lax.broadcasted_iota(jnp.int32, (B, 1), 0) == b
        return jnp.where(sel, jnp.maximum(cur, mrow), cur)

      lo = gids_ref[0, 0, 0]
      hi = gids_ref[0, 0, BLK - 1]
      mx_ref[...] = lax.fori_loop(lo, hi + 1, bstep, mx_ref[...])

  def call(agg, deg2, W, gid_row, gid_col, gid_sm):
    return pl.pallas_call(
        body,
        grid=(NB,),
        in_specs=[
            pl.BlockSpec((1, BLK, DQ), lambda i: (0, i, 0)),
            pl.BlockSpec((1, BLK, DQ), lambda i: (1, i, 0)),
            pl.BlockSpec((1, BLK, DQ), lambda i: (2, i, 0)),
            pl.BlockSpec((1, BLK, DQ), lambda i: (3, i, 0)),
            pl.BlockSpec((NCORE, BLK, 16), lambda i: (0, i, 0)),
            pl.BlockSpec((D, D), lambda i: (0, 0)),
            pl.BlockSpec((1, 1, BLK), lambda i: (i, 0, 0)),
            pl.BlockSpec((1, BLK, 1), lambda i: (i, 0, 0)),
            pl.BlockSpec((1, 1, BLK), lambda i: (i, 0, 0),
                         memory_space=pltpu.SMEM),
        ],
        out_specs=out_specs,
        out_shape=out_shape,
    )(agg, agg, agg, agg, deg2, W, gid_row, gid_col, gid_sm)

  return call


_layer_g1 = _make_layer(True, False)
_layer_g = _make_layer(False, False)
_layer_cg = _make_layer(False, True)


def _gather_rows(hh1, hh2, hh3, rel_emb, hidx, tidx, ridx):
  out_type = (
      jax.ShapeDtypeStruct((L, NQ, B, DQ), jnp.float32),
      jax.ShapeDtypeStruct((L, NQ, B, DQ), jnp.float32),
      jax.ShapeDtypeStruct((B, L * D), jnp.float32),
  )
  scratch = (
      pltpu.VMEM((B,), jnp.int32),
      pltpu.VMEM((B, DQ), jnp.float32),
      pltpu.VMEM((B, L * D), jnp.float32),
      pltpu.SemaphoreType.DMA,
  )

  def body(h1, h2, h3, rel_hbm, hidx_hbm, tidx_hbm, ridx_hbm,
           head_out, tail_out, rel_out, idx_v, buf, rbuf, sem):
    c = lax.axis_index("c")
    s = lax.axis_index("s")

    @pl.when(jnp.logical_and(c == 0, s == 0))
    def _():
      pltpu.sync_copy(hidx_hbm, idx_v)
      for l, h in enumerate((h1, h2, h3)):
        for qq in range(NQ):
          pltpu.async_copy(h.at[qq].at[idx_v], buf, sem).wait()
          pltpu.sync_copy(buf, head_out.at[l, qq])
      pltpu.sync_copy(tidx_hbm, idx_v)
      for l, h in enumerate((h1, h2, h3)):
        for qq in range(NQ):
          pltpu.async_copy(h.at[qq].at[idx_v], buf, sem).wait()
          pltpu.sync_copy(buf, tail_out.at[l, qq])
      pltpu.sync_copy(ridx_hbm, idx_v)
      pltpu.async_copy(rel_hbm.at[idx_v], rbuf, sem).wait()
      pltpu.sync_copy(rbuf, rel_out)

  return pl.kernel(body, out_type=out_type, mesh=_MESH,
                   scratch_types=scratch,
                   compiler_params=pltpu.CompilerParams(
                       use_tc_tiling_on_sc=False))(hh1, hh2, hh3, rel_emb,
                                                   hidx, tidx, ridx)


def _final(sgs, scgs, mcgs, cnt, head_r, tail_r, rel_r,
           fw_g, fw_head, fw_tail, fw_cg, fw_rel, fw_path, fcb):
  def body(sg1, sg2, sg3, sc1, sc2, sc3, mc1, mc2, mc3, cnt_ref,
           h_ref, t_ref, r_ref, wg_ref, wh_ref, wt_ref, wc_ref, wr_ref,
           wp_ref, b_ref, out_ref):
    cnt1 = jnp.maximum(cnt_ref[:, 0:1], 1.0)
    acc = jnp.dot(r_ref[...], wr_ref[...],
                  preferred_element_type=jnp.float32)
    for l, (sg, sc, mc) in enumerate(((sg1, sc1, mc1), (sg2, sc2, mc2),
                                      (sg3, sc3, mc3))):
      acc += jnp.dot(sg[...] / cnt1, wg_ref[l],
                     preferred_element_type=jnp.float32)
      acc += jnp.dot(sc[...] / cnt1, wc_ref[l],
                     preferred_element_type=jnp.float32)
      acc += jnp.dot(mc[...], wp_ref[l], preferred_element_type=jnp.float32)
    for k in range(L * NQ):
      acc += jnp.dot(h_ref[k], wh_ref[k], preferred_element_type=jnp.float32)
      acc += jnp.dot(t_ref[k], wt_ref[k], preferred_element_type=jnp.float32)
    out_ref[...] = acc + b_ref[...]

  return pl.pallas_call(
      body,
      out_shape=jax.ShapeDtypeStruct((B, 1), jnp.float32),
  )(*sgs, *scgs, *mcgs, cnt, head_r, tail_r, rel_r,
    fw_g, fw_head, fw_tail, fw_cg, fw_rel, fw_path, fcb)


def kernel(x, edge_index, x_cg, cg_edge_index, graph_ids, head_idx,
           tail_idx, rel_labels, W_g, W_cg, rel_emb, fc_W, fc_b):
  f32 = jnp.float32

  def quarters(xx):
    xp = jnp.concatenate([xx, jnp.zeros((NP - N, D), f32)], axis=0)
    return xp.reshape(NP, NQ, DQ).transpose(1, 0, 2)

  hhg0 = quarters(x)
  hhc0 = quarters(x_cg)

  def prep(eidx):
    src = jnp.concatenate([eidx[0], jnp.zeros((EPAD - E,), jnp.int32)])
    dst = jnp.concatenate([eidx[1], jnp.full((EPAD - E,), N, jnp.int32)])
    return src.reshape(NT, CPT, CH), dst.reshape(NT, CPT, CH)

  src_g, dst_g = prep(edge_index)
  src_c, dst_c = prep(cg_edge_index)

  zrows = jnp.zeros((ZCH, DQ), f32)
  z16 = jnp.zeros((ZCH, 16), f32)
  ones16 = jnp.ones((CH, 16), f32)

  gidp = jnp.concatenate([graph_ids, jnp.full((NP - N,), B, jnp.int32)])
  gid_row = gidp.reshape(NB, 1, BLK)
  gid_col = gidp.reshape(NB, BLK, 1)
  gid_sm = gidp.reshape(NB, 1, BLK)

  gg1, degg = _edge_pass_deg(hhg0, src_g, dst_g, zrows, ones16, z16)
  gc1, degc = _edge_pass_deg(hhc0, src_c, dst_c, zrows, ones16, z16)
  hhg1, sg1, cnt = _layer_g1(gg1, degg, W_g[0], gid_row, gid_col, gid_sm)
  hhc1, sc1, mc1 = _layer_cg(gc1, degc, W_cg[0], gid_row, gid_col, gid_sm)
  gg2, = _edge_pass(hhg1, src_g, dst_g, zrows)
  hhg2, sg2 = _layer_g(gg2, degg, W_g[1], gid_row, gid_col, gid_sm)
  gc2, = _edge_pass(hhc1, src_c, dst_c, zrows)
  hhc2, sc2, mc2 = _layer_cg(gc2, degc, W_cg[1], gid_row, gid_col, gid_sm)
  gg3, = _edge_pass(hhg2, src_g, dst_g, zrows)
  hhg3, sg3 = _layer_g(gg3, degg, W_g[2], gid_row, gid_col, gid_sm)
  gc3, = _edge_pass(hhc2, src_c, dst_c, zrows)
  _, sc3, mc3 = _layer_cg(gc3, degc, W_cg[2], gid_row, gid_col, gid_sm)

  head_r, tail_r, rel_r = _gather_rows(hhg1, hhg2, hhg3, rel_emb,
                                       head_idx, tail_idx, rel_labels)

  fw_g = fc_W[0:L * D].reshape(L, D, 1)
  fw_head = fc_W[L * D:2 * L * D].reshape(L * NQ, DQ, 1)
  fw_tail = fc_W[2 * L * D:3 * L * D].reshape(L * NQ, DQ, 1)
  fw_cg = fc_W[3 * L * D:4 * L * D].reshape(L, D, 1)
  fw_rel = fc_W[4 * L * D:5 * L * D]
  fw_path = fc_W[5 * L * D:6 * L * D].reshape(L, D, 1)
  fcb = fc_b.reshape(1, 1)

  return _final((sg1, sg2, sg3), (sc1, sc2, sc3), (mc1, mc2, mc3), cnt,
                head_r.reshape(L * NQ, B, DQ),
                tail_r.reshape(L * NQ, B, DQ), rel_r,
                fw_g, fw_head, fw_tail, fw_cg, fw_rel, fw_path, fcb)


# speedup vs baseline: 1.0383x; 1.0383x over previous
import jax
import jax.numpy as jnp
from jax import lax
from jax.experimental import pallas as pl
from jax.experimental.pallas import tpu as pltpu
from jax.experimental.pallas import tpu_sc as plsc

N = 10000
E = 320000
D = 128
L = 3
B = 64
R = 16

NCORE = 2
NQ = 4
DQ = D // NQ
DH = D // NCORE
NT = 16
CH = 128
CPT = 158
EPT = CH * CPT
EPAD = NT * EPT
NP = 10240
STRIPE = NP // NT
ZCH = 128

_MESH = plsc.VectorSubcoreMesh(core_axis_name="c", subcore_axis_name="s")


def _make_edge_pass(with_deg):
  out_type = [jax.ShapeDtypeStruct((NQ, NP, DQ), jnp.float32)]
  scratch = [
      pltpu.VMEM((CPT, CH), jnp.int32),
      pltpu.VMEM((CPT, CH), jnp.int32),
      pltpu.VMEM((CH, DQ), jnp.float32),
      pltpu.VMEM((CH, DQ), jnp.float32),
      pltpu.VMEM((ZCH, DQ), jnp.float32),
      pltpu.VMEM_SHARED((NP, DQ), jnp.float32),
      pltpu.VMEM_SHARED((NP, DQ), jnp.float32),
      pltpu.SemaphoreType.DMA,
      pltpu.SemaphoreType.DMA,
      pltpu.SemaphoreType.DMA,
      pltpu.SemaphoreType.DMA,
  ]
  if with_deg:
    out_type.append(jax.ShapeDtypeStruct((NCORE, NP, 16), jnp.float32))
    scratch += [
        pltpu.VMEM((CH, 16), jnp.float32),
        pltpu.VMEM((ZCH, 16), jnp.float32),
        pltpu.VMEM_SHARED((NP, 16), jnp.float32),
    ]

  def body(*args):
    if with_deg:
      (hq, src_idx, dst_idx, zrows, ones_hbm, z16_hbm,
       agg_out, deg_out,
       src_v, dst_v, rows0, rows1, zero_v, hsp, acc, g0, g1, s0, s1,
       ones_v, z16_v, dacc) = args
    else:
      (hq, src_idx, dst_idx, zrows,
       agg_out,
       src_v, dst_v, rows0, rows1, zero_v, hsp, acc, g0, g1, s0, s1) = args
    c = lax.axis_index("c")
    s = lax.axis_index("s")
    pltpu.sync_copy(zrows, zero_v)
    pltpu.sync_copy(src_idx.at[s], src_v)
    pltpu.sync_copy(dst_idx.at[s], dst_v)
    if with_deg:
      pltpu.sync_copy(ones_hbm, ones_v)
      pltpu.sync_copy(z16_hbm, z16_v)
    base = s * STRIPE
    for p in range(2):
      q = 2 * c + p
      pltpu.sync_copy(hq.at[q, pl.ds(base, STRIPE)],
                      hsp.at[pl.ds(base, STRIPE)])
      for k in range(STRIPE // ZCH):
        pltpu.sync_copy(zero_v, acc.at[pl.ds(base + k * ZCH, ZCH)])
        if with_deg and p == 0:
          pltpu.sync_copy(z16_v, dacc.at[pl.ds(base + k * ZCH, ZCH)])
      plsc.subcore_barrier()

      pltpu.async_copy(hsp.at[src_v.at[0]], rows0, g0)

      def pair(jj, carry):
        a = 2 * jj
        b = a + 1
        nxt = jnp.where(b + 1 >= CPT, 0, b + 1)
        pltpu.async_copy(hsp.at[src_v.at[b]], rows1, g1)
        pltpu.make_async_copy(hsp.at[src_v.at[a]], rows0, g0).wait()
        pltpu.async_copy(rows0, acc.at[dst_v.at[a]], s0, add=True)
        if with_deg and p == 0:
          @pl.when(c == 0)
          def _():
            pltpu.sync_copy(ones_v, dacc.at[dst_v.at[a]], add=True)

          @pl.when(c == 1)
          def _():
            pltpu.sync_copy(ones_v, dacc.at[dst_v.at[b]], add=True)
        pltpu.make_async_copy(hsp.at[src_v.at[b]], rows1, g1).wait()
        pltpu.async_copy(rows1, acc.at[dst_v.at[b]], s1, add=True)
        pltpu.make_async_copy(rows0, acc.at[dst_v.at[a]], s0).wait()
        pltpu.async_copy(hsp.at[src_v.at[nxt]], rows0, g0)
        pltpu.make_async_copy(rows1, acc.at[dst_v.at[b]], s1).wait()
        return carry

      lax.fori_loop(0, CPT // 2, pair, 0)
      pltpu.make_async_copy(hsp.at[src_v.at[0]], rows0, g0).wait()
      plsc.subcore_barrier()
      pltpu.sync_copy(acc.at[pl.ds(base, STRIPE)],
                      agg_out.at[q, pl.ds(base, STRIPE)])
      if with_deg and p == 0:
        pltpu.sync_copy(dacc.at[pl.ds(base, STRIPE)],
                        deg_out.at[c, pl.ds(base, STRIPE)])

  return pl.kernel(body, out_type=tuple(out_type), mesh=_MESH,
                   scratch_types=tuple(scratch),
                   compiler_params=pltpu.CompilerParams(
                       use_tc_tiling_on_sc=False))


_edge_pass_deg = _make_edge_pass(True)
_edge_pass = _make_edge_pass(False)


BLK = 512
NB = NP // BLK


def _make_layer(with_cnt, with_max):
  out_shape = [jax.ShapeDtypeStruct((NQ, NP, DQ), jnp.float32),
               jax.ShapeDtypeStruct((B, D), jnp.float32)]
  out_specs = [pl.BlockSpec((NQ, BLK, DQ), lambda i: (0, i, 0)),
               pl.BlockSpec((B, D), lambda i: (0, 0))]
  if with_cnt:
    out_shape.append(jax.ShapeDtypeStruct((B, 128), jnp.float32))
    out_specs.append(pl.BlockSpec((B, 128), lambda i: (0, 0)))
  if with_max:
    out_shape.append(jax.ShapeDtypeStruct((B, D), jnp.float32))
    out_specs.append(pl.BlockSpec((B, D), lambda i: (0, 0)))

  def body(*refs):
    (a0_ref, a1_ref, a2_ref, a3_ref, deg_ref, w_ref, gidr_ref, gidc_ref,
     gids_ref, hh_ref, sum_ref, *rest) = refs
    cnt_ref = rest[0] if with_cnt else None
    mx_ref = rest[1 if with_cnt else 0] if with_max else None

    i = pl.program_id(0)
    agg = jnp.concatenate([a0_ref[0], a1_ref[0], a2_ref[0], a3_ref[0]],
                          axis=1)
    deg = deg_ref[0, :, 0:1] + deg_ref[1, :, 0:1]
    norm = 1.0 / jnp.maximum(deg, 1.0)
    h = jnp.dot(agg * norm, w_ref[...], preferred_element_type=jnp.float32)
    h = jnp.maximum(h, 0.0)
    for q in range(NQ):
      hh_ref[q] = h[:, q * DQ:(q + 1) * DQ]

    @pl.when(i == 0)
    def _():
      sum_ref[...] = jnp.zeros_like(sum_ref)
      if with_cnt:
        cnt_ref[...] = jnp.zeros_like(cnt_ref)
      if with_max:
        mx_ref[...] = jnp.zeros_like(mx_ref)

    gid_row = gidr_ref[0]
    M = (gid_row == lax.broadcasted_iota(jnp.int32, (B, BLK), 0)
         ).astype(jnp.float32)
    sum_ref[...] += jnp.dot(M, h, preferred_element_type=jnp.float32)
    if with_cnt:
      cnt_ref[...] += jnp.dot(M, jnp.ones((BLK, 128), jnp.float32),
                              preferred_element_type=jnp.float32)
    if with_max:
      gid_col = gidc_ref[0]

      def bstep(b, cur):
        mrow = jnp.max(jnp.where(gid_col == b, h, 0.0), axis=0,
                       keepdims=True)
        sel = lax.broadcasted_iota(jnp.int32, (B, 1), 0) == b
        return jnp.where(sel, jnp.maximum(cur, mrow), cur)

      lo = gids_ref[0, 0, 0]
      hi = gids_ref[0, 0, BLK - 1]
      mx_ref[...] = lax.fori_loop(lo, hi + 1, bstep, mx_ref[...])

  def call(agg, deg2, W, gid_row, gid_col, gid_sm):
    return pl.pallas_call(
        body,
        grid=(NB,),
        in_specs=[
            pl.BlockSpec((1, BLK, DQ), lambda i: (0, i, 0)),
            pl.BlockSpec((1, BLK, DQ), lambda i: (1, i, 0)),
            pl.BlockSpec((1, BLK, DQ), lambda i: (2, i, 0)),
            pl.BlockSpec((1, BLK, DQ), lambda i: (3, i, 0)),
            pl.BlockSpec((NCORE, BLK, 16), lambda i: (0, i, 0)),
            pl.BlockSpec((D, D), lambda i: (0, 0)),
            pl.BlockSpec((1, 1, BLK), lambda i: (i, 0, 0)),
            pl.BlockSpec((1, BLK, 1), lambda i: (i, 0, 0)),
            pl.BlockSpec((1, 1, BLK), lambda i: (i, 0, 0),
                         memory_space=pltpu.SMEM),
        ],
        out_specs=out_specs,
        out_shape=out_shape,
    )(agg, agg, agg, agg, deg2, W, gid_row, gid_col, gid_sm)

  return call


_layer_g1 = _make_layer(True, False)
_layer_g = _make_layer(False, False)
_layer_cg = _make_layer(False, True)


def _gather_rows(hh1, hh2, hh3, rel_emb, hidx, tidx, ridx):
  out_type = (
      jax.ShapeDtypeStruct((L, NQ, B, DQ), jnp.float32),
      jax.ShapeDtypeStruct((L, NQ, B, DQ), jnp.float32),
      jax.ShapeDtypeStruct((B, L * D), jnp.float32),
  )
  scratch = (
      pltpu.VMEM((B,), jnp.int32),
      pltpu.VMEM((B, DQ), jnp.float32),
      pltpu.VMEM((B, L * D), jnp.float32),
      pltpu.SemaphoreType.DMA,
  )

  def body(h1, h2, h3, rel_hbm, hidx_hbm, tidx_hbm, ridx_hbm,
           head_out, tail_out, rel_out, idx_v, buf, rbuf, sem):
    c = lax.axis_index("c")
    s = lax.axis_index("s")

    @pl.when(jnp.logical_and(c == 0, s == 0))
    def _():
      pltpu.sync_copy(hidx_hbm, idx_v)
      for l, h in enumerate((h1, h2, h3)):
        for qq in range(NQ):
          pltpu.async_copy(h.at[qq].at[idx_v], buf, sem).wait()
          pltpu.sync_copy(buf, head_out.at[l, qq])
      pltpu.sync_copy(tidx_hbm, idx_v)
      for l, h in enumerate((h1, h2, h3)):
        for qq in range(NQ):
          pltpu.async_copy(h.at[qq].at[idx_v], buf, sem).wait()
          pltpu.sync_copy(buf, tail_out.at[l, qq])
      pltpu.sync_copy(ridx_hbm, idx_v)
      pltpu.async_copy(rel_hbm.at[idx_v], rbuf, sem).wait()
      pltpu.sync_copy(rbuf, rel_out)

  return pl.kernel(body, out_type=out_type, mesh=_MESH,
                   scratch_types=scratch,
                   compiler_params=pltpu.CompilerParams(
                       use_tc_tiling_on_sc=False))(hh1, hh2, hh3, rel_emb,
                                                   hidx, tidx, ridx)


def _final(sgs, scgs, mcgs, cnt, head_r, tail_r, rel_r,
           fw_g, fw_head, fw_tail, fw_cg, fw_rel, fw_path, fcb):
  def body(sg1, sg2, sg3, sc1, sc2, sc3, mc1, mc2, mc3, cnt_ref,
           h_ref, t_ref, r_ref, wg_ref, wh_ref, wt_ref, wc_ref, wr_ref,
           wp_ref, b_ref, out_ref):
    cnt1 = jnp.maximum(cnt_ref[:, 0:1], 1.0)
    acc = jnp.dot(r_ref[...], wr_ref[...],
                  preferred_element_type=jnp.float32)
    for l, (sg, sc, mc) in enumerate(((sg1, sc1, mc1), (sg2, sc2, mc2),
                                      (sg3, sc3, mc3))):
      acc += jnp.dot(sg[...] / cnt1, wg_ref[l],
                     preferred_element_type=jnp.float32)
      acc += jnp.dot(sc[...] / cnt1, wc_ref[l],
                     preferred_element_type=jnp.float32)
      acc += jnp.dot(mc[...], wp_ref[l], preferred_element_type=jnp.float32)
    for k in range(L * NQ):
      acc += jnp.dot(h_ref[k], wh_ref[k], preferred_element_type=jnp.float32)
      acc += jnp.dot(t_ref[k], wt_ref[k], preferred_element_type=jnp.float32)
    out_ref[...] = acc + b_ref[...]

  return pl.pallas_call(
      body,
      out_shape=jax.ShapeDtypeStruct((B, 1), jnp.float32),
  )(*sgs, *scgs, *mcgs, cnt, head_r, tail_r, rel_r,
    fw_g, fw_head, fw_tail, fw_cg, fw_rel, fw_path, fcb)


def kernel(x, edge_index, x_cg, cg_edge_index, graph_ids, head_idx,
           tail_idx, rel_labels, W_g, W_cg, rel_emb, fc_W, fc_b):
  f32 = jnp.float32

  def quarters(xx):
    xp = jnp.concatenate([xx, jnp.zeros((NP - N, D), f32)], axis=0)
    return xp.reshape(NP, NQ, DQ).transpose(1, 0, 2)

  hhg0 = quarters(x)
  hhc0 = quarters(x_cg)

  pad_rows = N + jnp.arange(EPAD - E, dtype=jnp.int32) % (NP - N)

  def prep(eidx):
    src = jnp.concatenate([eidx[0], pad_rows])
    dst = jnp.concatenate([eidx[1], pad_rows])
    return src.reshape(NT, CPT, CH), dst.reshape(NT, CPT, CH)

  src_g, dst_g = prep(edge_index)
  src_c, dst_c = prep(cg_edge_index)

  zrows = jnp.zeros((ZCH, DQ), f32)
  z16 = jnp.zeros((ZCH, 16), f32)
  ones16 = jnp.ones((CH, 16), f32)

  gidp = jnp.concatenate([graph_ids, jnp.full((NP - N,), B, jnp.int32)])
  gid_row = gidp.reshape(NB, 1, BLK)
  gid_col = gidp.reshape(NB, BLK, 1)
  gid_sm = gidp.reshape(NB, 1, BLK)

  gg1, degg = _edge_pass_deg(hhg0, src_g, dst_g, zrows, ones16, z16)
  gc1, degc = _edge_pass_deg(hhc0, src_c, dst_c, zrows, ones16, z16)
  hhg1, sg1, cnt = _layer_g1(gg1, degg, W_g[0], gid_row, gid_col, gid_sm)
  hhc1, sc1, mc1 = _layer_cg(gc1, degc, W_cg[0], gid_row, gid_col, gid_sm)
  gg2, = _edge_pass(hhg1, src_g, dst_g, zrows)
  hhg2, sg2 = _layer_g(gg2, degg, W_g[1], gid_row, gid_col, gid_sm)
  gc2, = _edge_pass(hhc1, src_c, dst_c, zrows)
  hhc2, sc2, mc2 = _layer_cg(gc2, degc, W_cg[1], gid_row, gid_col, gid_sm)
  gg3, = _edge_pass(hhg2, src_g, dst_g, zrows)
  hhg3, sg3 = _layer_g(gg3, degg, W_g[2], gid_row, gid_col, gid_sm)
  gc3, = _edge_pass(hhc2, src_c, dst_c, zrows)
  _, sc3, mc3 = _layer_cg(gc3, degc, W_cg[2], gid_row, gid_col, gid_sm)

  head_r, tail_r, rel_r = _gather_rows(hhg1, hhg2, hhg3, rel_emb,
                                       head_idx, tail_idx, rel_labels)

  fw_g = fc_W[0:L * D].reshape(L, D, 1)
  fw_head = fc_W[L * D:2 * L * D].reshape(L * NQ, DQ, 1)
  fw_tail = fc_W[2 * L * D:3 * L * D].reshape(L * NQ, DQ, 1)
  fw_cg = fc_W[3 * L * D:4 * L * D].reshape(L, D, 1)
  fw_rel = fc_W[4 * L * D:5 * L * D]
  fw_path = fc_W[5 * L * D:6 * L * D].reshape(L, D, 1)
  fcb = fc_b.reshape(1, 1)

  return _final((sg1, sg2, sg3), (sc1, sc2, sc3), (mc1, mc2, mc3), cnt,
                head_r.reshape(L * NQ, B, DQ),
                tail_r.reshape(L * NQ, B, DQ), rel_r,
                fw_g, fw_head, fw_tail, fw_cg, fw_rel, fw_path, fcb)
